# baseline (speedup 1.0000x reference)
import functools

import jax
import jax.numpy as jnp
from jax import lax
from jax.experimental import pallas as pl
from jax.experimental.pallas import tpu as pltpu
from jax.experimental.pallas import tpu_sc as plsc

_NUM_CLASSES = 100000
_EMBED_DIM = 32
_MULT = 2
_B, _L = 4096, 200
_N = _B * _L * _MULT

_NC, _NS, _LANES = 2, 16, 16
_NW = _NC * _NS
_DMA_ROWS = 128
_NROWS = _N // _DMA_ROWS
_ROWS_W = _NROWS // _NW
_K = 8
_CHUNK = _K * _DMA_ROWS
_NCHUNK = _ROWS_W // _K
_DT = _EMBED_DIM // 8
_BT = _B // _DMA_ROWS

_mesh = plsc.VectorSubcoreMesh(core_axis_name="c", subcore_axis_name="s")


@functools.partial(
    pl.kernel,
    out_type=jax.ShapeDtypeStruct((_L, _MULT, _DT, _BT, 8, 128), jnp.float32),
    mesh=_mesh,
    scratch_types=[
        pltpu.VMEM((2, _K, _DMA_ROWS), jnp.int32),
        pltpu.VMEM((2, _CHUNK, _EMBED_DIM), jnp.float32),
        pltpu.VMEM((4, _DT, 8, 129), jnp.float32),
        [pltpu.SemaphoreType.DMA] * 2,
        [pltpu.SemaphoreType.DMA] * 2,
        [pltpu.SemaphoreType.DMA] * 4,
    ],
    compiler_params=pltpu.CompilerParams(
        use_tc_tiling_on_sc=False, needs_layout_passes=False
    ),
)
def _gather(feat_hbm, table_hbm, out_hbm, idx_v, rows_v, stage_v, si, sg, st):
    wid = lax.axis_index("s") * _NC + lax.axis_index("c")
    row_base = wid * _ROWS_W
    iota = lax.iota(jnp.int32, 16)
    di_ix = iota & 7
    dt_ix = [lax.shift_right_logical(iota, 3) + 2 * h for h in range(2)]

    def add_offsets(b):
        for j in range(_K):
            if j & 1:
                r = idx_v.at[b].at[j]
                for i in range(_DMA_ROWS // _LANES):
                    sl = pl.ds(i * _LANES, _LANES)
                    r[sl] = r[sl] + _NUM_CLASSES

    def fire_gathers(b):
        for j in range(_K):
            pltpu.async_copy(
                table_hbm.at[idx_v.at[b].at[j]],
                rows_v.at[b].at[pl.ds(j * _DMA_ROWS, _DMA_ROWS)],
                sg[b],
            )

    def issue_idx(c, b):
        pltpu.async_copy(feat_hbm.at[pl.ds(row_base + c * _K, _K)],
                         idx_v.at[b], si[b])

    def wait_idx(b):
        pltpu.make_async_copy(feat_hbm.at[pl.ds(0, _K)], idx_v.at[b],
                              si[b]).wait()

    def wait_gathers(b):
        pltpu.make_async_copy(table_hbm.at[pl.ds(0, _CHUNK)], rows_v.at[b],
                              sg[b]).wait()

    def transpose_chunk(g, b):
        rb = row_base + g * _K

        def row_quad(jj, _):
            for p in range(4):
                j = 4 * jj + p
                m = p & 1
                row = rb + j
                l_ = lax.shift_right_logical(row, 6)
                bt_ = lax.shift_right_logical(row, 1) & (_BT - 1)
                rbase = j * _DMA_ROWS

                @pl.when(g * _K + j >= 4)
                def _wait_prev():
                    for dt in range(_DT):
                        pltpu.make_async_copy(
                            stage_v.at[p, dt, :, pl.ds(0, 128)],
                            out_hbm.at[0, 0, 0, 0],
                            st[p],
                        ).wait()

                stg = stage_v.at[p]
                for bq in range(0, _DMA_ROWS, 8):
                    vecs = [
                        rows_v[b, rbase + bq + k, pl.ds(h * 16, 16)]
                        for k in range(8) for h in range(2)
                    ]
                    for k in range(8):
                        bi_ix = jnp.full((16,), bq + k, jnp.int32)
                        for h in range(2):
                            plsc.store_scatter(
                                stg, [dt_ix[h], di_ix, bi_ix],
                                vecs[2 * k + h],
                            )
                for dt in range(_DT):
                    pltpu.async_copy(
                        stage_v.at[p, dt, :, pl.ds(0, 128)],
                        out_hbm.at[l_, m, dt, bt_],
                        st[p],
                    )
            return ()

        lax.fori_loop(0, _K // 4, row_quad, ())

    issue_idx(0, 0)
    wait_idx(0)
    add_offsets(0)
    fire_gathers(0)
    issue_idx(1, 1)

    def body(t, _):
        for parity in range(2):
            g = 2 * t + parity
            bg, bn = parity, 1 - parity

            @pl.when(g < _NCHUNK - 1)
            def _fire_next():
                wait_idx(bn)
                add_offsets(bn)
                fire_gathers(bn)

            wait_gathers(bg)

            @pl.when(g < _NCHUNK - 2)
            def _issue_next_idx():
                issue_idx(g + 2, bg)

            transpose_chunk(g, bg)
        return ()

    lax.fori_loop(0, _NCHUNK // 2, body, ())

    for p in range(4):
        for dt in range(_DT):
            pltpu.make_async_copy(
                stage_v.at[p, dt, :, pl.ds(0, 128)], out_hbm.at[0, 0, 0, 0],
                st[p],
            ).wait()


def kernel(feat, table):
    fv = feat.reshape(_BT, _DMA_ROWS, _L, _MULT).transpose(2, 0, 3, 1)
    out6 = _gather(fv.reshape(_NROWS, _DMA_ROWS), table)
    out = out6.transpose(3, 5, 0, 1, 2, 4)
    return out.reshape(_B, _L, _MULT, _EMBED_DIM)


# speedup vs baseline: 1.3313x; 1.3313x over previous
import functools

import jax
import jax.numpy as jnp
from jax import lax
from jax.experimental import pallas as pl
from jax.experimental.pallas import tpu as pltpu
from jax.experimental.pallas import tpu_sc as plsc

_NUM_CLASSES = 100000
_EMBED_DIM = 32
_MULT = 2
_B, _L = 4096, 200
_N = _B * _L * _MULT

_NC, _NS, _LANES = 2, 16, 16
_NW = _NC * _NS
_DMA_ROWS = 128
_NROWS = _N // _DMA_ROWS
_ROWS_W = _NROWS // _NW
_K = 10
_CHUNK = _K * _DMA_ROWS
_NCHUNK = _ROWS_W // _K
_DT = _EMBED_DIM // 8
_BT = _B // _DMA_ROWS

_mesh = plsc.VectorSubcoreMesh(core_axis_name="c", subcore_axis_name="s")


@functools.partial(
    pl.kernel,
    out_type=jax.ShapeDtypeStruct((_L, _MULT, _DT, _BT, 8, 128), jnp.float32),
    mesh=_mesh,
    scratch_types=[
        pltpu.VMEM((2, _K, _DMA_ROWS), jnp.int32),
        pltpu.VMEM((2, _CHUNK, _EMBED_DIM), jnp.float32),
        pltpu.VMEM((2, _DT, 8, 129), jnp.float32),
        [pltpu.SemaphoreType.DMA] * 2,
        [pltpu.SemaphoreType.DMA] * 2,
        [pltpu.SemaphoreType.DMA] * 2,
    ],
    compiler_params=pltpu.CompilerParams(
        use_tc_tiling_on_sc=False, needs_layout_passes=False
    ),
)
def _gather(feat_hbm, table_hbm, out_hbm, idx_v, rows_v, stage_v, si, sg, st):
    wid = lax.axis_index("s") * _NC + lax.axis_index("c")
    row_base = wid * _ROWS_W
    iota = lax.iota(jnp.int32, 16)
    di_ix = iota & 7
    dt_ix = [lax.shift_right_logical(iota, 3) + 2 * h for h in range(2)]

    def add_offsets(b):
        for j in range(_K):
            if j & 1:
                r = idx_v.at[b].at[j]
                for i in range(_DMA_ROWS // _LANES):
                    sl = pl.ds(i * _LANES, _LANES)
                    r[sl] = r[sl] + _NUM_CLASSES

    def fire_gathers(b):
        for j in range(_K):
            pltpu.async_copy(
                table_hbm.at[idx_v.at[b].at[j]],
                rows_v.at[b].at[pl.ds(j * _DMA_ROWS, _DMA_ROWS)],
                sg[b],
            )

    def issue_idx(c, b):
        pltpu.async_copy(feat_hbm.at[pl.ds(row_base + c * _K, _K)],
                         idx_v.at[b], si[b])

    def wait_idx(b):
        pltpu.make_async_copy(feat_hbm.at[pl.ds(0, _K)], idx_v.at[b],
                              si[b]).wait()

    def wait_gathers(b):
        pltpu.make_async_copy(table_hbm.at[pl.ds(0, _CHUNK)], rows_v.at[b],
                              sg[b]).wait()

    def transpose_chunk(g, b):
        rb = row_base + g * _K

        def row_pair(jj, _):
            for p in range(2):
                j = 2 * jj + p
                row = rb + j
                l_ = lax.shift_right_logical(row, 6)
                bt_ = lax.shift_right_logical(row, 1) & (_BT - 1)
                rbase = j * _DMA_ROWS

                @pl.when(g * _K + j >= 2)
                def _wait_prev():
                    for dt in range(_DT):
                        pltpu.make_async_copy(
                            stage_v.at[p, dt, :, pl.ds(0, 128)],
                            out_hbm.at[0, 0, 0, 0],
                            st[p],
                        ).wait()

                stg = stage_v.at[p]
                for bq in range(0, _DMA_ROWS, 8):
                    vecs = [
                        rows_v[b, rbase + bq + k, pl.ds(h * 16, 16)]
                        for k in range(8) for h in range(2)
                    ]
                    for k in range(8):
                        bi_ix = jnp.full((16,), bq + k, jnp.int32)
                        for h in range(2):
                            plsc.store_scatter(
                                stg, [dt_ix[h], di_ix, bi_ix],
                                vecs[2 * k + h],
                            )
                for dt in range(_DT):
                    pltpu.async_copy(
                        stage_v.at[p, dt, :, pl.ds(0, 128)],
                        out_hbm.at[l_, p, dt, bt_],
                        st[p],
                    )
            return ()

        lax.fori_loop(0, _K // 2, row_pair, ())

    issue_idx(0, 0)
    wait_idx(0)
    add_offsets(0)
    fire_gathers(0)
    issue_idx(1, 1)

    def body(t, _):
        for parity in range(2):
            g = 2 * t + parity
            bg, bn = parity, 1 - parity

            @pl.when(g < _NCHUNK - 1)
            def _fire_next():
                wait_idx(bn)
                add_offsets(bn)
                fire_gathers(bn)

            wait_gathers(bg)

            @pl.when(g < _NCHUNK - 2)
            def _issue_next_idx():
                issue_idx(g + 2, bg)

            transpose_chunk(g, bg)
        return ()

    lax.fori_loop(0, _NCHUNK // 2, body, ())

    for p in range(2):
        for dt in range(_DT):
            pltpu.make_async_copy(
                stage_v.at[p, dt, :, pl.ds(0, 128)], out_hbm.at[0, 0, 0, 0],
                st[p],
            ).wait()


def kernel(feat, table):
    fv = feat.reshape(_BT, _DMA_ROWS, _L, _MULT).transpose(2, 0, 3, 1)
    out6 = _gather(fv.reshape(_NROWS, _DMA_ROWS), table)
    out = out6.transpose(3, 5, 0, 1, 2, 4)
    return out.reshape(_B, _L, _MULT, _EMBED_DIM)
